# baseline (speedup 1.0000x reference)
import functools

import jax
import jax.numpy as jnp
from jax import lax
from jax.experimental import pallas as pl
from jax.experimental.pallas import tpu as pltpu
from jax.experimental.pallas import tpu_sc as plsc

B = 16384
H = 64
W = 2 * H
NE = 1000000
NR = 1000
L = 16
NC = 2
NS = 16
NW = NC * NS
BPW = B // NW
CB = 64
NCHUNK = BPW // CB
NBUF = 2
NGROUP = CB // L
KH = H // L

CW = 2048
NBLK = 245
HALFP = NBLK * CW

def _relayout_tc_body(ia, ib, o):
    o[:, 0:H] = ia[...].T
    o[:, H:W] = ib[...].T


_relayout_tc = pl.pallas_call(
    _relayout_tc_body,
    grid=(NBLK,),
    in_specs=[pl.BlockSpec((H, CW), lambda c: (0, c)),
              pl.BlockSpec((H, CW), lambda c: (0, jnp.minimum(c + NBLK, NE // CW)))],
    out_specs=pl.BlockSpec((CW, W), lambda c: (c, 0)),
    out_shape=jax.ShapeDtypeStruct((HALFP, W), jnp.float32),
)

NCOL = NE // W
TAILE = NE - NCOL * W
SP = 65
RBUF = 2


def _relayout_sc_body(eiT, eip, vin, vcmp, vtl, vsc, sem_i, sem_o):
    wid = lax.axis_index("s") * NC + lax.axis_index("c")
    c65 = lax.iota(jnp.int32, L) * SP
    nk = (NCOL + NW - 1) // NW

    def col(k):
        return wid + k * NW

    def issue_in(k, s):
        pltpu.async_copy(eiT.at[:, pl.ds(col(k) * W, W)], vin.at[s], sem_i)

    def wait_in(k, s):
        pltpu.make_async_copy(eiT.at[:, pl.ds(col(k) * W, W)], vin.at[s],
                              sem_i).wait()

    def issue_out(k, s):
        pltpu.async_copy(vcmp.at[s], eip.at[pl.ds(col(k) * H, H)], sem_o)

    def wait_out(k, s):
        pltpu.make_async_copy(vcmp.at[s], eip.at[pl.ds(col(k) * H, H)],
                              sem_o).wait()

    def xpose(vin_ref):
        def h_body(h, _):
            for lb in range(W // L):
                v = vin_ref[h, pl.ds(lb * L, L)]
                plsc.store_scatter(vsc, [c65 + (h + lb * (L * SP))], v)
            return 0

        lax.fori_loop(0, H, h_body, 0)

    if True:
        @pl.when(col(0) < NCOL)
        def _():
            issue_in(0, 0)

        def k_body(k, _):
            for s in range(RBUF):
                kk = k * RBUF + s

                @pl.when(col(kk + 1) < NCOL)
                def _():
                    issue_in(kk + 1, (s + 1) % RBUF)

                @pl.when(col(kk) < NCOL)
                def _():
                    wait_in(kk, s)
                    xpose(vin.at[s])

                    @pl.when(kk >= RBUF)
                    def _():
                        wait_out(kk - RBUF, s)

                    def p_body(p, _):
                        for j, off in enumerate((0, 16, 32, 48, 65, 81, 97, 113)):
                            vcmp[s, p, pl.ds(j * L, L)] = (
                                vsc[pl.ds(p * (2 * SP) + off, L)])
                        return 0

                    lax.fori_loop(0, H, p_body, 0)
                    issue_out(kk, s)
            return 0

        lax.fori_loop(0, (nk + RBUF - 1) // RBUF, k_body, 0)
        for kk in (nk - 3, nk - 2, nk - 1):
            @pl.when((col(kk) < NCOL) & (col(kk + RBUF) >= NCOL))
            def _(kk=kk):
                wait_out(kk, kk % RBUF)

        @pl.when(wid == 0)
        def _():
            pltpu.async_copy(eiT.at[:, pl.ds(NCOL * W, TAILE)], vtl, sem_i).wait()

            def ht_body(h, _):
                for lb in range(TAILE // L):
                    v = vtl[h, pl.ds(lb * L, L)]
                    plsc.store_scatter(vsc, [c65 + (h + lb * (L * SP))], v)
                return 0

            lax.fori_loop(0, H, ht_body, 0)

            def pt_body(p, _):
                for j, off in enumerate((0, 16, 32, 48, 65, 81, 97, 113)):
                    vcmp[0, p, pl.ds(j * L, L)] = vsc[pl.ds(p * (2 * SP) + off, L)]
                return 0

            lax.fori_loop(0, TAILE // 2, pt_body, 0)
            pltpu.async_copy(vcmp.at[0, pl.ds(0, TAILE // 2)],
                             eip.at[pl.ds(NCOL * H, TAILE // 2)], sem_i).wait()


_relayout_sc = functools.partial(
    pl.kernel,
    out_type=jax.ShapeDtypeStruct((NE // 2, W), jnp.float32),
    mesh=plsc.VectorSubcoreMesh(core_axis_name="c", subcore_axis_name="s"),
    compiler_params=pltpu.CompilerParams(needs_layout_passes=False),
    scratch_types=[
        pltpu.VMEM((RBUF, H, W), jnp.float32),
        pltpu.VMEM((RBUF, H, W), jnp.float32),
        pltpu.VMEM((H, TAILE), jnp.float32),
        pltpu.VMEM((H * 2 * SP,), jnp.float32),
        pltpu.SemaphoreType.DMA,
        pltpu.SemaphoreType.DMA,
    ],
)(_relayout_sc_body)


def _score_body(src_h, rel_h, dst_h, er_h, ei_h, rr_h, ri_h, out_h,
                s_raw, r_raw, d_raw, s_row, r_row, d_row, sp_row, dp_row,
                sr_b, si_b, dr_b, di_b, qr_b, qi_b,
                p_v, out_v, sem0, sem1):
    sems = (sem0, sem1)
    wid = lax.axis_index("s") * NC + lax.axis_index("c")
    base = wid * BPW

    stage = []
    for c in range(NCHUNK):
        off = base + c * CB
        stage.append(pltpu.async_copy(src_h.at[pl.ds(off, CB)], s_raw.at[c], sem0))
        stage.append(pltpu.async_copy(rel_h.at[pl.ds(off, CB)], r_raw.at[c], sem0))
        stage.append(pltpu.async_copy(dst_h.at[pl.ds(off, CB)], d_raw.at[c], sem0))
    for cp in stage:
        cp.wait()

    for c in range(NCHUNK):
        for g in range(NGROUP):
            ds = pl.ds(g * L, L)
            sv = s_raw[c, ds]
            dv = d_raw[c, ds]
            s_row[c, ds] = jnp.where(sv < HALFP, sv, sv - HALFP)
            d_row[c, ds] = jnp.where(dv < HALFP, dv, dv - HALFP)
            sp_row[c, ds] = lax.shift_right_logical(sv, 1)
            dp_row[c, ds] = lax.shift_right_logical(dv, 1)
            r_row[c, ds] = lax.shift_right_logical(r_raw[c, ds], 1)

    gathers = ((er_h, s_row, sr_b), (ei_h, sp_row, si_b),
               (er_h, d_row, dr_b), (ei_h, dp_row, di_b),
               (rr_h, r_row, qr_b), (ri_h, r_row, qi_b))

    def issue(cc, slot):
        for tab, rref, buf in gathers:
            pltpu.async_copy(tab.at[rref.at[cc]], buf.at[slot], sems[slot])

    def drain(cc, slot):
        for tab, rref, buf in gathers:
            pltpu.make_async_copy(tab.at[rref.at[cc]], buf.at[slot],
                                  sems[slot]).wait()

    def compute(cc, slot):
        def g_body(g, _):
            sv = s_raw[cc, pl.ds(g * L, L)]
            rv = r_raw[cc, pl.ds(g * L, L)]
            dv = d_raw[cc, pl.ds(g * L, L)]
            for el in range(L):
                e = g * L + el
                so = jnp.where(sv[el] < HALFP, 0, H)
                soi = (sv[el] & 1) * H
                ro = (rv[el] & 1) * H
                do = jnp.where(dv[el] < HALFP, 0, H)
                doi = (dv[el] & 1) * H
                acc = jnp.zeros((L,), jnp.float32)
                for k in range(KH):
                    a = sr_b[slot, e, pl.ds(so + k * L, L)]
                    bi = si_b[slot, e, pl.ds(soi + k * L, L)]
                    cr = dr_b[slot, e, pl.ds(do + k * L, L)]
                    ci = di_b[slot, e, pl.ds(doi + k * L, L)]
                    rr = qr_b[slot, e, pl.ds(ro + k * L, L)]
                    ri = qi_b[slot, e, pl.ds(ro + k * L, L)]
                    acc = acc + rr * (a * cr + bi * ci) + ri * (a * ci - bi * cr)
                p_v[pl.ds(el * (L + 1), L)] = acc
            rows = lax.iota(jnp.int32, L) * (L + 1)
            tot = jnp.zeros((L,), jnp.float32)
            for j in range(L):
                tot = tot + plsc.load_gather(p_v, [rows + j])
            out_v[pl.ds(cc * CB + g * L, L)] = tot
            return 0

        lax.fori_loop(0, NGROUP, g_body, 0)

    issue(0, 0)

    def pipe_body(it, _):
        for b in range(NBUF):
            cc = it * NBUF + b
            nxt = cc + 1

            @pl.when(nxt < NCHUNK)
            def _():
                issue(nxt, (b + 1) % NBUF)

            drain(cc, b)
            compute(cc, b)
        return 0

    lax.fori_loop(0, NCHUNK // NBUF, pipe_body, 0)

    pltpu.sync_copy(out_v, out_h.at[pl.ds(base, BPW)])


_score = functools.partial(
    pl.kernel,
    out_type=jax.ShapeDtypeStruct((B,), jnp.float32),
    mesh=plsc.VectorSubcoreMesh(core_axis_name="c", subcore_axis_name="s"),
    compiler_params=pltpu.CompilerParams(needs_layout_passes=False),
    scratch_types=[
        pltpu.VMEM((NCHUNK, CB), jnp.int32),
        pltpu.VMEM((NCHUNK, CB), jnp.int32),
        pltpu.VMEM((NCHUNK, CB), jnp.int32),
        pltpu.VMEM((NCHUNK, CB), jnp.int32),
        pltpu.VMEM((NCHUNK, CB), jnp.int32),
        pltpu.VMEM((NCHUNK, CB), jnp.int32),
        pltpu.VMEM((NCHUNK, CB), jnp.int32),
        pltpu.VMEM((NCHUNK, CB), jnp.int32),
        pltpu.VMEM((NBUF, CB, W), jnp.float32),
        pltpu.VMEM((NBUF, CB, W), jnp.float32),
        pltpu.VMEM((NBUF, CB, W), jnp.float32),
        pltpu.VMEM((NBUF, CB, W), jnp.float32),
        pltpu.VMEM((NBUF, CB, W), jnp.float32),
        pltpu.VMEM((NBUF, CB, W), jnp.float32),
        pltpu.VMEM((L * (L + 1),), jnp.float32),
        pltpu.VMEM((BPW,), jnp.float32),
        pltpu.SemaphoreType.DMA,
        pltpu.SemaphoreType.DMA,
    ],
)(_score_body)


@jax.jit
def kernel(src, rel, dst, ent_real, ent_imag, rel_real, rel_imag):
    er_p = _relayout_tc(ent_real.T, ent_real.T)
    ei_p = _relayout_sc(ent_imag.T)
    return _score(src.astype(jnp.int32), rel.astype(jnp.int32),
                  dst.astype(jnp.int32), er_p, ei_p,
                  rel_real.reshape(NR // 2, W),
                  rel_imag.reshape(NR // 2, W))


# speedup vs baseline: 1.7233x; 1.7233x over previous
import functools

import jax
import jax.numpy as jnp
from jax import lax
from jax.experimental import pallas as pl
from jax.experimental.pallas import tpu as pltpu
from jax.experimental.pallas import tpu_sc as plsc

B = 16384
H = 64
W = 2 * H
NE = 1000000
NR = 1000
L = 16
NC = 2
NS = 16
NW = NC * NS
BPW = B // NW
CB = 64
NCHUNK = BPW // CB
NBUF = 2
NGROUP = CB // L
KH = H // L

CW = 2048
NBLK = 245
HALFP = NBLK * CW

def _relayout_tc_body(ia1, ib1, ia2, ib2, o1, o2):
    o1[:, 0:H] = ia1[...].T
    o1[:, H:W] = ib1[...].T
    o2[:, 0:H] = ia2[...].T
    o2[:, H:W] = ib2[...].T


_relayout_tc = pl.pallas_call(
    _relayout_tc_body,
    grid=(NBLK,),
    in_specs=[pl.BlockSpec((H, CW), lambda c: (0, c)),
              pl.BlockSpec((H, CW), lambda c: (0, jnp.minimum(c + NBLK, NE // CW))),
              pl.BlockSpec((H, CW), lambda c: (0, c)),
              pl.BlockSpec((H, CW), lambda c: (0, jnp.minimum(c + NBLK, NE // CW)))],
    out_specs=[pl.BlockSpec((CW, W), lambda c: (c, 0)),
               pl.BlockSpec((CW, W), lambda c: (c, 0))],
    out_shape=(jax.ShapeDtypeStruct((HALFP, W), jnp.float32),
               jax.ShapeDtypeStruct((HALFP, W), jnp.float32)),
)


def _score_body(src_h, rel_h, dst_h, er_h, ei_h, rr_h, ri_h, out_h,
                s_raw, r_raw, d_raw, s_row, r_row, d_row,
                sr_b, si_b, dr_b, di_b, qr_b, qi_b,
                p_v, out_v, sem0, sem1):
    sems = (sem0, sem1)
    wid = lax.axis_index("s") * NC + lax.axis_index("c")
    base = wid * BPW

    stage = []
    for c in range(NCHUNK):
        off = base + c * CB
        stage.append(pltpu.async_copy(src_h.at[pl.ds(off, CB)], s_raw.at[c], sem0))
        stage.append(pltpu.async_copy(rel_h.at[pl.ds(off, CB)], r_raw.at[c], sem0))
        stage.append(pltpu.async_copy(dst_h.at[pl.ds(off, CB)], d_raw.at[c], sem0))
    for cp in stage:
        cp.wait()

    for c in range(NCHUNK):
        for g in range(NGROUP):
            ds = pl.ds(g * L, L)
            sv = s_raw[c, ds]
            dv = d_raw[c, ds]
            s_row[c, ds] = jnp.where(sv < HALFP, sv, sv - HALFP)
            d_row[c, ds] = jnp.where(dv < HALFP, dv, dv - HALFP)
            r_row[c, ds] = lax.shift_right_logical(r_raw[c, ds], 1)

    gathers = ((er_h, s_row, sr_b), (ei_h, s_row, si_b),
               (er_h, d_row, dr_b), (ei_h, d_row, di_b),
               (rr_h, r_row, qr_b), (ri_h, r_row, qi_b))

    def issue(cc, slot):
        for tab, rref, buf in gathers:
            pltpu.async_copy(tab.at[rref.at[cc]], buf.at[slot], sems[slot])

    def drain(cc, slot):
        for tab, rref, buf in gathers:
            pltpu.make_async_copy(tab.at[rref.at[cc]], buf.at[slot],
                                  sems[slot]).wait()

    def compute(cc, slot):
        def g_body(g, _):
            sv = s_raw[cc, pl.ds(g * L, L)]
            rv = r_raw[cc, pl.ds(g * L, L)]
            dv = d_raw[cc, pl.ds(g * L, L)]
            for el in range(L):
                e = g * L + el
                so = jnp.where(sv[el] < HALFP, 0, H)
                ro = (rv[el] & 1) * H
                do = jnp.where(dv[el] < HALFP, 0, H)
                acc = jnp.zeros((L,), jnp.float32)
                for k in range(KH):
                    a = sr_b[slot, e, pl.ds(so + k * L, L)]
                    bi = si_b[slot, e, pl.ds(so + k * L, L)]
                    cr = dr_b[slot, e, pl.ds(do + k * L, L)]
                    ci = di_b[slot, e, pl.ds(do + k * L, L)]
                    rr = qr_b[slot, e, pl.ds(ro + k * L, L)]
                    ri = qi_b[slot, e, pl.ds(ro + k * L, L)]
                    acc = acc + rr * (a * cr + bi * ci) + ri * (a * ci - bi * cr)
                p_v[pl.ds(el * (L + 1), L)] = acc
            rows = lax.iota(jnp.int32, L) * (L + 1)
            tot = jnp.zeros((L,), jnp.float32)
            for j in range(L):
                tot = tot + plsc.load_gather(p_v, [rows + j])
            out_v[pl.ds(cc * CB + g * L, L)] = tot
            return 0

        lax.fori_loop(0, NGROUP, g_body, 0)

    issue(0, 0)

    def pipe_body(it, _):
        for b in range(NBUF):
            cc = it * NBUF + b
            nxt = cc + 1

            @pl.when(nxt < NCHUNK)
            def _():
                issue(nxt, (b + 1) % NBUF)

            drain(cc, b)
            compute(cc, b)
        return 0

    lax.fori_loop(0, NCHUNK // NBUF, pipe_body, 0)

    pltpu.sync_copy(out_v, out_h.at[pl.ds(base, BPW)])


_score = functools.partial(
    pl.kernel,
    out_type=jax.ShapeDtypeStruct((B,), jnp.float32),
    mesh=plsc.VectorSubcoreMesh(core_axis_name="c", subcore_axis_name="s"),
    compiler_params=pltpu.CompilerParams(needs_layout_passes=False),
    scratch_types=[
        pltpu.VMEM((NCHUNK, CB), jnp.int32),
        pltpu.VMEM((NCHUNK, CB), jnp.int32),
        pltpu.VMEM((NCHUNK, CB), jnp.int32),
        pltpu.VMEM((NCHUNK, CB), jnp.int32),
        pltpu.VMEM((NCHUNK, CB), jnp.int32),
        pltpu.VMEM((NCHUNK, CB), jnp.int32),
        pltpu.VMEM((NBUF, CB, W), jnp.float32),
        pltpu.VMEM((NBUF, CB, W), jnp.float32),
        pltpu.VMEM((NBUF, CB, W), jnp.float32),
        pltpu.VMEM((NBUF, CB, W), jnp.float32),
        pltpu.VMEM((NBUF, CB, W), jnp.float32),
        pltpu.VMEM((NBUF, CB, W), jnp.float32),
        pltpu.VMEM((L * (L + 1),), jnp.float32),
        pltpu.VMEM((BPW,), jnp.float32),
        pltpu.SemaphoreType.DMA,
        pltpu.SemaphoreType.DMA,
    ],
)(_score_body)


@jax.jit
def kernel(src, rel, dst, ent_real, ent_imag, rel_real, rel_imag):
    er_p, ei_p = _relayout_tc(ent_real.T, ent_real.T, ent_imag.T, ent_imag.T)
    return _score(src.astype(jnp.int32), rel.astype(jnp.int32),
                  dst.astype(jnp.int32), er_p, ei_p,
                  rel_real.reshape(NR // 2, W),
                  rel_imag.reshape(NR // 2, W))


# speedup vs baseline: 2.0351x; 1.1810x over previous
import functools

import jax
import jax.numpy as jnp
from jax import lax
from jax.experimental import pallas as pl
from jax.experimental.pallas import tpu as pltpu
from jax.experimental.pallas import tpu_sc as plsc

B = 16384
H = 64
W = 2 * H
NE = 1000000
NR = 1000
L = 16
NC = 2
NS = 16
NW = NC * NS
BPW = B // NW
CB = 64
NCHUNK = BPW // CB
NBUF = 2
NGROUP = CB // L
KH = H // L

CW = 8192
NBLK = 62
HALFP = NBLK * CW

def _relayout_tc_body(ia1, ib1, ia2, ib2, o1, o2):
    o1[:, 0:H] = ia1[...].T
    o1[:, H:W] = ib1[...].T
    o2[:, 0:H] = ia2[...].T
    o2[:, H:W] = ib2[...].T


_relayout_tc = pl.pallas_call(
    _relayout_tc_body,
    grid=(NBLK,),
    in_specs=[pl.BlockSpec((H, CW), lambda c: (0, c)),
              pl.BlockSpec((H, CW), lambda c: (0, jnp.minimum(c + NBLK, NE // CW))),
              pl.BlockSpec((H, CW), lambda c: (0, c)),
              pl.BlockSpec((H, CW), lambda c: (0, jnp.minimum(c + NBLK, NE // CW)))],
    out_specs=[pl.BlockSpec((CW, W), lambda c: (c, 0)),
               pl.BlockSpec((CW, W), lambda c: (c, 0))],
    out_shape=(jax.ShapeDtypeStruct((HALFP, W), jnp.float32),
               jax.ShapeDtypeStruct((HALFP, W), jnp.float32)),
)


def _score_body(src_h, rel_h, dst_h, er_h, ei_h, rr_h, ri_h, out_h,
                s_raw, r_raw, d_raw, s_row, r_row, d_row,
                sr_b, si_b, dr_b, di_b, qr_b, qi_b,
                p_v, out_v, sem0, sem1):
    sems = (sem0, sem1)
    wid = lax.axis_index("s") * NC + lax.axis_index("c")
    base = wid * BPW

    stage = []
    for c in range(NCHUNK):
        off = base + c * CB
        stage.append(pltpu.async_copy(src_h.at[pl.ds(off, CB)], s_raw.at[c], sem0))
        stage.append(pltpu.async_copy(rel_h.at[pl.ds(off, CB)], r_raw.at[c], sem0))
        stage.append(pltpu.async_copy(dst_h.at[pl.ds(off, CB)], d_raw.at[c], sem0))
    for cp in stage:
        cp.wait()

    for c in range(NCHUNK):
        for g in range(NGROUP):
            ds = pl.ds(g * L, L)
            sv = s_raw[c, ds]
            dv = d_raw[c, ds]
            s_row[c, ds] = jnp.where(sv < HALFP, sv, sv - HALFP)
            d_row[c, ds] = jnp.where(dv < HALFP, dv, dv - HALFP)
            r_row[c, ds] = lax.shift_right_logical(r_raw[c, ds], 1)

    gathers = ((er_h, s_row, sr_b), (ei_h, s_row, si_b),
               (er_h, d_row, dr_b), (ei_h, d_row, di_b),
               (rr_h, r_row, qr_b), (ri_h, r_row, qi_b))

    def issue(cc, slot):
        for tab, rref, buf in gathers:
            pltpu.async_copy(tab.at[rref.at[cc]], buf.at[slot], sems[slot])

    def drain(cc, slot):
        for tab, rref, buf in gathers:
            pltpu.make_async_copy(tab.at[rref.at[cc]], buf.at[slot],
                                  sems[slot]).wait()

    def compute(cc, slot):
        def g_body(g, _):
            sv = s_raw[cc, pl.ds(g * L, L)]
            rv = r_raw[cc, pl.ds(g * L, L)]
            dv = d_raw[cc, pl.ds(g * L, L)]
            for el in range(L):
                e = g * L + el
                so = jnp.where(sv[el] < HALFP, 0, H)
                ro = (rv[el] & 1) * H
                do = jnp.where(dv[el] < HALFP, 0, H)
                acc = jnp.zeros((L,), jnp.float32)
                for k in range(KH):
                    a = sr_b[slot, e, pl.ds(so + k * L, L)]
                    bi = si_b[slot, e, pl.ds(so + k * L, L)]
                    cr = dr_b[slot, e, pl.ds(do + k * L, L)]
                    ci = di_b[slot, e, pl.ds(do + k * L, L)]
                    rr = qr_b[slot, e, pl.ds(ro + k * L, L)]
                    ri = qi_b[slot, e, pl.ds(ro + k * L, L)]
                    acc = acc + rr * (a * cr + bi * ci) + ri * (a * ci - bi * cr)
                p_v[pl.ds(el * (L + 1), L)] = acc
            rows = lax.iota(jnp.int32, L) * (L + 1)
            tot = jnp.zeros((L,), jnp.float32)
            for j in range(L):
                tot = tot + plsc.load_gather(p_v, [rows + j])
            out_v[pl.ds(cc * CB + g * L, L)] = tot
            return 0

        lax.fori_loop(0, NGROUP, g_body, 0)

    issue(0, 0)

    def pipe_body(it, _):
        for b in range(NBUF):
            cc = it * NBUF + b
            nxt = cc + 1

            @pl.when(nxt < NCHUNK)
            def _():
                issue(nxt, (b + 1) % NBUF)

            drain(cc, b)
            compute(cc, b)
        return 0

    lax.fori_loop(0, NCHUNK // NBUF, pipe_body, 0)

    pltpu.sync_copy(out_v, out_h.at[pl.ds(base, BPW)])


_score = functools.partial(
    pl.kernel,
    out_type=jax.ShapeDtypeStruct((B,), jnp.float32),
    mesh=plsc.VectorSubcoreMesh(core_axis_name="c", subcore_axis_name="s"),
    compiler_params=pltpu.CompilerParams(needs_layout_passes=False),
    scratch_types=[
        pltpu.VMEM((NCHUNK, CB), jnp.int32),
        pltpu.VMEM((NCHUNK, CB), jnp.int32),
        pltpu.VMEM((NCHUNK, CB), jnp.int32),
        pltpu.VMEM((NCHUNK, CB), jnp.int32),
        pltpu.VMEM((NCHUNK, CB), jnp.int32),
        pltpu.VMEM((NCHUNK, CB), jnp.int32),
        pltpu.VMEM((NBUF, CB, W), jnp.float32),
        pltpu.VMEM((NBUF, CB, W), jnp.float32),
        pltpu.VMEM((NBUF, CB, W), jnp.float32),
        pltpu.VMEM((NBUF, CB, W), jnp.float32),
        pltpu.VMEM((NBUF, CB, W), jnp.float32),
        pltpu.VMEM((NBUF, CB, W), jnp.float32),
        pltpu.VMEM((L * (L + 1),), jnp.float32),
        pltpu.VMEM((BPW,), jnp.float32),
        pltpu.SemaphoreType.DMA,
        pltpu.SemaphoreType.DMA,
    ],
)(_score_body)


@jax.jit
def kernel(src, rel, dst, ent_real, ent_imag, rel_real, rel_imag):
    er_p, ei_p = _relayout_tc(ent_real.T, ent_real.T, ent_imag.T, ent_imag.T)
    return _score(src.astype(jnp.int32), rel.astype(jnp.int32),
                  dst.astype(jnp.int32), er_p, ei_p,
                  rel_real.reshape(NR // 2, W),
                  rel_imag.reshape(NR // 2, W))
